# baseline (speedup 1.0000x reference)
import jax
import jax.numpy as jnp
from jax import lax
from jax.experimental import pallas as pl
from jax.experimental.pallas import tpu as pltpu
from jax.experimental.pallas import tpu_sc as plsc

NC = 2
NS = 16
NW = NC * NS
D = 64
CH = 128


def _body(x_hbm, w_hbm, out_hbm, idx_v, rows_v, gsem):
    t_per_w = x_hbm.shape[1]
    wid = lax.axis_index("s") * NC + lax.axis_index("c")
    per_w = t_per_w * CH
    pltpu.sync_copy(x_hbm.at[wid], idx_v)

    @pl.loop(0, t_per_w)
    def _(t):
        off = wid * per_w + t * CH
        pltpu.async_copy(w_hbm.at[idx_v.at[t]], rows_v, gsem).wait()
        pltpu.sync_copy(rows_v, out_hbm.at[pl.ds(off, CH)])


def kernel(x, weight):
    b0, b1 = x.shape
    rows = b0 * b1
    assert rows % (NW * CH) == 0
    t_per_w = rows // (NW * CH)
    xr = x.reshape(NW, t_per_w, CH).astype(jnp.int32)

    mesh = plsc.VectorSubcoreMesh(core_axis_name="c", subcore_axis_name="s")
    out = pl.kernel(
        _body,
        out_type=jax.ShapeDtypeStruct((rows, D), jnp.float32),
        mesh=mesh,
        scratch_types=[
            pltpu.VMEM((t_per_w, CH), jnp.int32),
            pltpu.VMEM((CH, D), jnp.float32),
            pltpu.SemaphoreType.DMA,
        ],
        compiler_params=pltpu.CompilerParams(use_tc_tiling_on_sc=False),
    )(xr, weight)
    return out.reshape(b0, b1, D)


# speedup vs baseline: 1.1119x; 1.1119x over previous
import jax
import jax.numpy as jnp
from jax import lax
from jax.experimental import pallas as pl
from jax.experimental.pallas import tpu as pltpu
from jax.experimental.pallas import tpu_sc as plsc

NC = 2
NS = 16
NW = NC * NS
D = 64
CH = 128


GCH = 4
GR = GCH * CH


def _body(x_hbm, w_hbm, out_hbm, idx_v, buf0, buf1, g0, g1, s0, s1):
    t_per_w = x_hbm.shape[1]
    ngrp = t_per_w // GCH
    wid = lax.axis_index("s") * NC + lax.axis_index("c")
    woff = wid * t_per_w * CH
    pltpu.sync_copy(x_hbm.at[wid], idx_v)

    def fire_gathers(grp, buf, sem):
        for j in range(GCH):
            pltpu.async_copy(
                w_hbm.at[idx_v.at[grp * GCH + j]],
                buf.at[pl.ds(j * CH, CH)],
                sem,
            )

    def wait_gathers(buf, sem):
        for j in range(GCH):
            pltpu.make_async_copy(
                w_hbm.at[idx_v.at[j]], buf.at[pl.ds(j * CH, CH)], sem
            ).wait()

    fire_gathers(0, buf0, g0)
    fire_gathers(1, buf1, g1)

    @pl.loop(0, ngrp // 2)
    def _(i):
        ga = 2 * i
        gb = 2 * i + 1
        wait_gathers(buf0, g0)
        sa = pltpu.async_copy(buf0, out_hbm.at[pl.ds(woff + ga * GR, GR)], s0)
        wait_gathers(buf1, g1)
        sb = pltpu.async_copy(buf1, out_hbm.at[pl.ds(woff + gb * GR, GR)], s1)
        sa.wait()

        @pl.when(ga + 2 < ngrp)
        def _():
            fire_gathers(ga + 2, buf0, g0)

        sb.wait()

        @pl.when(gb + 2 < ngrp)
        def _():
            fire_gathers(gb + 2, buf1, g1)


def kernel(x, weight):
    b0, b1 = x.shape
    rows = b0 * b1
    assert rows % (NW * CH) == 0
    t_per_w = rows // (NW * CH)
    xr = x.reshape(NW, t_per_w, CH).astype(jnp.int32)

    mesh = plsc.VectorSubcoreMesh(core_axis_name="c", subcore_axis_name="s")
    out = pl.kernel(
        _body,
        out_type=jax.ShapeDtypeStruct((rows, D), jnp.float32),
        mesh=mesh,
        scratch_types=[
            pltpu.VMEM((t_per_w, CH), jnp.int32),
            pltpu.VMEM((GR, D), jnp.float32),
            pltpu.VMEM((GR, D), jnp.float32),
            pltpu.SemaphoreType.DMA,
            pltpu.SemaphoreType.DMA,
            pltpu.SemaphoreType.DMA,
            pltpu.SemaphoreType.DMA,
        ],
        compiler_params=pltpu.CompilerParams(use_tc_tiling_on_sc=False),
    )(xr, weight)
    return out.reshape(b0, b1, D)
